# baseline (speedup 1.0000x reference)
import functools

import jax
import jax.numpy as jnp
from jax import lax
from jax.experimental import pallas as pl
from jax.experimental.pallas import tpu as pltpu
from jax.experimental.pallas import tpu_sc as plsc


def _order_map_sc(B, N, D, I):
    info = plsc.get_sparse_core_info()
    NC, NS, L = info.num_cores, info.num_subcores, info.num_lanes
    NW = NC * NS
    total = B * I
    per_w = total // NW
    assert total % NW == 0 and per_w % L == 0 and I % per_w == 0 and N % 2 == 0

    mesh = plsc.VectorSubcoreMesh(core_axis_name="c", subcore_axis_name="s")

    @functools.partial(
        pl.kernel,
        mesh=mesh,
        out_type=jax.ShapeDtypeStruct((total, D), jnp.float32),
        scratch_types=[
            pltpu.VMEM((per_w,), jnp.int32),
            pltpu.VMEM((per_w,), jnp.int32),
            pltpu.VMEM((per_w,), jnp.int32),
            pltpu.VMEM((per_w, 2 * D), jnp.float32),
            pltpu.VMEM((per_w, D), jnp.float32),
            pltpu.SemaphoreType.DMA,
        ],
        compiler_params=pltpu.CompilerParams(needs_layout_passes=False),
    )
    def gather_kernel(x_hbm, idx_hbm, out_hbm, idx_v, pair_v, half_v,
                      grp_v, rows_v, sem):
        wid = lax.axis_index("s") * NC + lax.axis_index("c")
        row0 = wid * per_w
        b = row0 // I
        i0 = row0 - b * I
        pltpu.sync_copy(idx_hbm.at[pl.ds(i0, per_w)], idx_v)
        base = b * N
        for j in range(per_w // L):
            v = idx_v[pl.ds(j * L, L)]
            flat = jnp.minimum(jnp.maximum(v, 0), N - 1) + base
            pair_v[pl.ds(j * L, L)] = lax.shift_right_logical(flat, 1)
            half_v[pl.ds(j * L, L)] = lax.bitwise_and(flat, 1) * D
        pltpu.async_copy(x_hbm.at[pair_v], grp_v, sem).wait()
        lanes = lax.iota(jnp.int32, L)
        for g in range(per_w // L):
            row16 = lanes + g * L
            off16 = half_v[pl.ds(g * L, L)]
            for e in range(D):
                e16 = jnp.full((L,), e, jnp.int32)
                vals = plsc.load_gather(grp_v, [row16, off16 + e16])
                plsc.store_scatter(rows_v, [row16, e16], vals)
        pltpu.sync_copy(rows_v, out_hbm.at[pl.ds(row0, per_w)])

    return gather_kernel


def kernel(x, indices):
    B, N, D = x.shape
    I = indices.shape[0]
    xf = x.astype(jnp.float32)
    out = _order_map_sc(B, N, D, I)(xf.reshape(B * N // 2, 2 * D), indices)
    return out.reshape(B, I, D)


# speedup vs baseline: 2.6263x; 2.6263x over previous
import functools

import jax
import jax.numpy as jnp
from jax import lax
from jax.experimental import pallas as pl
from jax.experimental.pallas import tpu as pltpu
from jax.experimental.pallas import tpu_sc as plsc


def _order_map_sc(B, N, D, I):
    info = plsc.get_sparse_core_info()
    NC, NS, L = info.num_cores, info.num_subcores, info.num_lanes
    NW = NC * NS
    total = B * I
    per_w = total // NW
    n_chunks = 2
    chunk = per_w // n_chunks
    assert total % NW == 0 and chunk % L == 0 and I % per_w == 0 and N % 8 == 0

    mesh = plsc.VectorSubcoreMesh(core_axis_name="c", subcore_axis_name="s")

    @functools.partial(
        pl.kernel,
        mesh=mesh,
        out_type=jax.ShapeDtypeStruct((total, D), jnp.float32),
        scratch_types=[
            pltpu.VMEM((per_w,), jnp.int32),
            pltpu.VMEM((per_w,), jnp.int32),
            pltpu.VMEM((per_w,), jnp.int32),
            pltpu.VMEM((chunk, 8, D), jnp.float32),
            pltpu.VMEM((per_w, D), jnp.float32),
            pltpu.SemaphoreType.DMA,
        ],
        compiler_params=pltpu.CompilerParams(needs_layout_passes=False),
    )
    def gather_kernel(x_hbm, idx_hbm, out_hbm, idx_v, grp_id_v, sub_v,
                      grp_v, rows_v, sem):
        wid = lax.axis_index("s") * NC + lax.axis_index("c")
        row0 = wid * per_w
        b = row0 // I
        i0 = row0 - b * I
        pltpu.sync_copy(idx_hbm.at[pl.ds(i0, per_w)], idx_v)
        base = b * N
        for j in range(per_w // L):
            v = idx_v[pl.ds(j * L, L)]
            flat = jnp.minimum(jnp.maximum(v, 0), N - 1) + base
            grp_id_v[pl.ds(j * L, L)] = lax.shift_right_logical(flat, 3)
            sub_v[pl.ds(j * L, L)] = lax.bitwise_and(flat, 7)
        lanes = lax.iota(jnp.int32, L)
        for c in range(n_chunks):
            copies = []
            for g in range(chunk // L):
                gvec = grp_id_v[pl.ds(c * chunk + g * L, L)]
                for l in range(L):
                    i = g * L + l
                    copies.append(
                        pltpu.async_copy(x_hbm.at[gvec[l]], grp_v.at[i], sem))
            for cp in copies:
                cp.wait()

            def select_body(e, carry):
                e16 = jnp.full((L,), e, jnp.int32)
                for g in range(chunk // L):
                    row16 = lanes + g * L
                    r16 = sub_v[pl.ds(c * chunk + g * L, L)]
                    vals = plsc.load_gather(grp_v, [row16, r16, e16])
                    plsc.store_scatter(
                        rows_v, [row16 + c * chunk, e16], vals)
                return carry

            lax.fori_loop(0, D, select_body, 0)
        pltpu.sync_copy(rows_v, out_hbm.at[pl.ds(row0, per_w)])

    return gather_kernel


def kernel(x, indices):
    B, N, D = x.shape
    I = indices.shape[0]
    xf = x.astype(jnp.float32)
    out = _order_map_sc(B, N, D, I)(xf.reshape(B * N // 8, 8, D), indices)
    return out.reshape(B, I, D)


# speedup vs baseline: 7.2804x; 2.7721x over previous
import functools

import jax
import jax.numpy as jnp
from jax import lax
from jax.experimental import pallas as pl
from jax.experimental.pallas import tpu as pltpu
from jax.experimental.pallas import tpu_sc as plsc


def _order_map_sc(B, N, D, I):
    info = plsc.get_sparse_core_info()
    NC, NS, L = info.num_cores, info.num_subcores, info.num_lanes
    NW = NC * NS
    TPB = N // 128
    DB = D // 8
    K = D // 2
    CI = 128 // K
    n_chunks = I // CI
    assert NW == 2 * B and D % 16 == 0 and N % 128 == 0
    assert 128 % K == 0 and I % CI == 0 and n_chunks % 2 == 0 and L == 16

    mesh = plsc.VectorSubcoreMesh(core_axis_name="c", subcore_axis_name="s")

    @functools.partial(
        pl.kernel,
        mesh=mesh,
        out_type=jax.ShapeDtypeStruct((B * D, I), jnp.float32),
        scratch_types=[
            pltpu.VMEM((I,), jnp.int32),
            pltpu.VMEM((I,), jnp.int32),
            pltpu.VMEM((I,), jnp.int32),
            pltpu.VMEM((128,), jnp.int32),
            pltpu.VMEM((n_chunks, 128), jnp.int32),
            pltpu.VMEM((128, 128), jnp.float32),
            pltpu.VMEM((128, 128), jnp.float32),
            pltpu.VMEM((K, I), jnp.float32),
            pltpu.SemaphoreType.DMA,
            pltpu.SemaphoreType.DMA,
        ],
        compiler_params=pltpu.CompilerParams(needs_layout_passes=False),
    )
    def gather_kernel(z_hbm, idx_hbm, out_hbm, idx_v, tc_v, lane_v, pat_v,
                      ids_v, grp0_v, grp1_v, out_v, sem0, sem1):
        wid = lax.axis_index("s") * NC + lax.axis_index("c")
        b = lax.shift_right_logical(wid, 1)
        db0 = lax.bitwise_and(wid, 1) * (DB // 2)
        lanes = lax.iota(jnp.int32, L)

        pltpu.sync_copy(idx_hbm, idx_v)
        for j in range(I // L):
            v = idx_v[pl.ds(j * L, L)]
            c = jnp.minimum(jnp.maximum(v, 0), N - 1)
            tc_v[pl.ds(j * L, L)] = lax.shift_right_logical(c, 7)
            lane_v[pl.ds(j * L, L)] = lax.bitwise_and(c, 127)

        for j in range(128 // L):
            k16 = lanes + (j * L) % K
            rb16 = b * DB + db0 + lax.shift_right_logical(k16, 3)
            pat_v[pl.ds(j * L, L)] = rb16 * (TPB * 8) + lax.bitwise_and(k16, 7)

        def build_ids(cidx, carry):
            for j in range(128 // L):
                i16 = jnp.full((L,), j * L // K, jnp.int32) + cidx * CI
                tc16 = plsc.load_gather(tc_v, [i16])
                ids_v[cidx, pl.ds(j * L, L)] = (
                    pat_v[pl.ds(j * L, L)] + tc16 * 8)
            return carry
        lax.fori_loop(0, n_chunks, build_ids, 0)

        def issue(cidx, grp, sem):
            pltpu.async_copy(z_hbm.at[ids_v.at[cidx]], grp, sem)

        def drain(cidx, grp, sem):
            pltpu.make_async_copy(z_hbm.at[ids_v.at[cidx]], grp, sem).wait()

        def select(cidx, grp):
            i16 = cidx * CI + lax.bitwise_and(lanes, CI - 1)
            cols16 = plsc.load_gather(lane_v, [i16])
            for kk in range(K * CI // L):
                k16 = kk * (L // CI) + lax.shift_right_logical(lanes, 2)
                r16 = lax.bitwise_and(lanes, CI - 1) * K + k16
                vals = plsc.load_gather(grp, [r16, cols16])
                plsc.store_scatter(out_v, [k16, i16], vals)

        issue(0, grp0_v, sem0)

        def body(it, carry):
            c0 = it * 2
            issue(c0 + 1, grp1_v, sem1)
            drain(c0, grp0_v, sem0)
            select(c0, grp0_v)

            @pl.when(c0 + 2 < n_chunks)
            def _():
                issue(c0 + 2, grp0_v, sem0)

            drain(c0 + 1, grp1_v, sem1)
            select(c0 + 1, grp1_v)
            return carry
        lax.fori_loop(0, n_chunks // 2, body, 0)

        pltpu.sync_copy(out_v, out_hbm.at[pl.ds(wid * K, K)])

    return gather_kernel


def kernel(x, indices):
    B, N, D = x.shape
    I = indices.shape[0]
    xf = x.astype(jnp.float32)
    xt = jnp.transpose(xf, (0, 2, 1))
    z = (xt.reshape(B * D // 8, 8, N // 128, 128)
           .transpose(0, 2, 1, 3)
           .reshape(B * D * N // 128, 128))
    out_t = _order_map_sc(B, N, D, I)(z, indices)
    return jnp.transpose(out_t.reshape(B, D, I), (0, 2, 1))


# speedup vs baseline: 16.7985x; 2.3074x over previous
import functools

import jax
import jax.numpy as jnp
from jax import lax
from jax.experimental import pallas as pl
from jax.experimental.pallas import tpu as pltpu
from jax.experimental.pallas import tpu_sc as plsc


def _order_map_sc(B, N, D, I):
    info = plsc.get_sparse_core_info()
    NC, NS, L = info.num_cores, info.num_subcores, info.num_lanes
    NW = NC * NS
    TPB = N // 128
    DB = D // 8
    K = D // 2
    CI = 128 // K
    n_chunks = I // CI
    n_rounds = 4
    rc = n_chunks // n_rounds
    assert NW == 2 * B and D == 64 and N % 128 == 0 and L == 16
    assert I % CI == 0 and n_chunks % n_rounds == 0

    mesh = plsc.VectorSubcoreMesh(core_axis_name="c", subcore_axis_name="s")

    @functools.partial(
        pl.kernel,
        mesh=mesh,
        out_type=jax.ShapeDtypeStruct((NW * D, 128), jnp.float32),
        scratch_types=[
            pltpu.VMEM((I,), jnp.int32),
            pltpu.VMEM((I,), jnp.int32),
            pltpu.VMEM((I,), jnp.int32),
            pltpu.VMEM((128,), jnp.int32),
            pltpu.VMEM((n_chunks, 128), jnp.int32),
            pltpu.VMEM((rc, 128, 16), jnp.float32),
            pltpu.VMEM((rc, 128, 16), jnp.float32),
            pltpu.VMEM((2 * K, 128), jnp.float32),
            pltpu.SemaphoreType.DMA,
            pltpu.SemaphoreType.DMA,
        ],
        compiler_params=pltpu.CompilerParams(
            needs_layout_passes=False, use_tc_tiling_on_sc=False),
    )
    def gather_kernel(z_hbm, idx_hbm, out_hbm, idx_v, g_v, lane_v, pat_v,
                      ids_v, grp0_v, grp1_v, out_v, sem0, sem1):
        wid = lax.axis_index("s") * NC + lax.axis_index("c")
        b = lax.shift_right_logical(wid, 1)
        db0 = lax.bitwise_and(wid, 1) * (DB // 2)
        lanes = lax.iota(jnp.int32, L)

        pltpu.sync_copy(idx_hbm, idx_v)
        for j in range(I // L):
            v = idx_v[pl.ds(j * L, L)]
            c = jnp.minimum(jnp.maximum(v, 0), N - 1)
            g_v[pl.ds(j * L, L)] = (
                lax.shift_right_logical(c, 7) * 64
                + lax.bitwise_and(lax.shift_right_logical(c, 4), 7))
            lane_v[pl.ds(j * L, L)] = lax.bitwise_and(c, 15)

        for j in range(128 // L):
            k16 = lanes + (j * L) % K
            rb16 = b * DB + db0 + lax.shift_right_logical(k16, 3)
            pat_v[pl.ds(j * L, L)] = (
                rb16 * (TPB * 64) + lax.bitwise_and(k16, 7) * 8)

        def build_ids(cidx, carry):
            for j in range(128 // L):
                i16 = jnp.full((L,), j * L // K, jnp.int32) + cidx * CI
                g16 = plsc.load_gather(g_v, [i16])
                ids_v[cidx, pl.ds(j * L, L)] = pat_v[pl.ds(j * L, L)] + g16
            return carry
        lax.fori_loop(0, n_chunks, build_ids, 0)

        grps = (grp0_v, grp1_v)
        sems = (sem0, sem1)

        def issue_round(r):
            for jj in range(rc):
                pltpu.async_copy(z_hbm.at[ids_v.at[r * rc + jj]],
                                 grps[r % 2].at[jj], sems[r % 2])

        def drain_round(r):
            for jj in range(rc):
                pltpu.make_async_copy(z_hbm.at[ids_v.at[r * rc + jj]],
                                      grps[r % 2].at[jj],
                                      sems[r % 2]).wait()

        def select_round(r):
            grp = grps[r % 2]

            def body(jj, carry):
                i16 = (r * rc + jj) * CI + lax.bitwise_and(lanes, CI - 1)
                cols16 = plsc.load_gather(lane_v, [i16])
                jj16 = jnp.full((L,), 0, jnp.int32) + jj
                for kk in range(K * CI // L):
                    k16 = kk * (L // CI) + lax.shift_right_logical(lanes, 2)
                    r16 = lax.bitwise_and(lanes, CI - 1) * K + k16
                    vals = plsc.load_gather(grp, [jj16, r16, cols16])
                    row16 = (lax.shift_right_logical(k16, 3) * 16
                             + lax.shift_right_logical(i16, 7) * 8
                             + lax.bitwise_and(k16, 7))
                    plsc.store_scatter(
                        out_v, [row16, lax.bitwise_and(i16, 127)], vals)
                return carry
            lax.fori_loop(0, rc, body, 0)

        issue_round(0)
        for r in range(n_rounds):
            if r + 1 < n_rounds:
                issue_round(r + 1)
            drain_round(r)
            select_round(r)

        pltpu.sync_copy(out_v, out_hbm.at[pl.ds(wid * (2 * K), 2 * K)])

    return gather_kernel


def kernel(x, indices):
    B, N, D = x.shape
    I = indices.shape[0]
    NW = 2 * B
    xf = x.astype(jnp.float32)
    xt = jnp.transpose(xf, (0, 2, 1))
    z = (xt.reshape(B * D // 8, 8, N // 128, 128)
           .transpose(0, 2, 1, 3)
           .reshape(B * D * N // 16, 16))
    out_p = _order_map_sc(B, N, D, I)(z, indices)
    out_t = (out_p.reshape(B * D // 8, 2, 8, 128)
                  .transpose(0, 2, 1, 3)
                  .reshape(B * D, I))
    return jnp.transpose(out_t.reshape(B, D, I), (0, 2, 1))
